# Initial kernel scaffold; baseline (speedup 1.0000x reference)
import jax
import jax.numpy as jnp
from jax.experimental import pallas as pl


def kernel(x, W1, b1, W2, b2, class_type_map):
    raise NotImplementedError("write your pallas kernel here")


# speedup vs baseline: 1.1003x; 1.1003x over previous
import functools

import jax
import jax.numpy as jnp
from jax.experimental import pallas as pl

B = 4096
D = 1024
C = 1000
T = 100
BM = 512


def _fused_kernel(x_ref, w1_ref, b1_ref, w2_ref, b2_ref, ctm_ref,
                  final_ref, cls_ref, type_ref):
    x = x_ref[...]

    l2 = jnp.dot(x, w2_ref[...], preferred_element_type=jnp.float32)
    l2 = l2 + b2_ref[...]
    e2 = jnp.exp(l2 - jnp.max(l2, axis=1, keepdims=True))
    out_type = e2 / jnp.sum(e2, axis=1, keepdims=True)
    type_ref[...] = out_type

    l1 = jnp.dot(x, w1_ref[...], preferred_element_type=jnp.float32)
    l1 = l1 + b1_ref[...]
    e1 = jnp.exp(l1 - jnp.max(l1, axis=1, keepdims=True))
    out_cls = e1 / jnp.sum(e1, axis=1, keepdims=True)
    cls_ref[...] = out_cls

    ctm = ctm_ref[...]
    tid = jax.lax.broadcasted_iota(jnp.int32, (T, C), 0)
    g = (ctm == tid).astype(jnp.float32)
    ctw = jnp.dot(out_type, g, preferred_element_type=jnp.float32)
    final_ref[...] = out_cls * (ctw + 1e-8)


@functools.partial(jax.jit, static_argnames=())
def kernel(x, W1, b1, W2, b2, class_type_map):
    b1r = b1.reshape(1, C)
    b2r = b2.reshape(1, T)
    ctm = class_type_map.reshape(1, C)
    grid = (B // BM,)
    out = pl.pallas_call(
        _fused_kernel,
        grid=grid,
        in_specs=[
            pl.BlockSpec((BM, D), lambda i: (i, 0)),
            pl.BlockSpec((D, C), lambda i: (0, 0)),
            pl.BlockSpec((1, C), lambda i: (0, 0)),
            pl.BlockSpec((D, T), lambda i: (0, 0)),
            pl.BlockSpec((1, T), lambda i: (0, 0)),
            pl.BlockSpec((1, C), lambda i: (0, 0)),
        ],
        out_specs=[
            pl.BlockSpec((BM, C), lambda i: (i, 0)),
            pl.BlockSpec((BM, C), lambda i: (i, 0)),
            pl.BlockSpec((BM, T), lambda i: (i, 0)),
        ],
        out_shape=[
            jax.ShapeDtypeStruct((B, C), jnp.float32),
            jax.ShapeDtypeStruct((B, C), jnp.float32),
            jax.ShapeDtypeStruct((B, T), jnp.float32),
        ],
    )(x, W1, b1r, W2, b2r, ctm)
    return (out[0], out[1], out[2])


# speedup vs baseline: 1.1023x; 1.0018x over previous
import functools

import jax
import jax.numpy as jnp
from jax.experimental import pallas as pl

B = 4096
D = 1024
C = 1000
T = 100
BM = 512


def _fused_kernel(x_ref, w1_ref, b1_ref, w2_ref, b2_ref, ctm_ref,
                  final_ref, cls_ref, type_ref):
    x = x_ref[...].astype(jnp.bfloat16)

    l2 = jnp.dot(x, w2_ref[...].astype(jnp.bfloat16),
                 preferred_element_type=jnp.float32)
    l2 = l2 + b2_ref[...]
    e2 = jnp.exp(l2 - jnp.max(l2, axis=1, keepdims=True))
    out_type = e2 / jnp.sum(e2, axis=1, keepdims=True)
    type_ref[...] = out_type

    l1 = jnp.dot(x, w1_ref[...].astype(jnp.bfloat16),
                 preferred_element_type=jnp.float32)
    l1 = l1 + b1_ref[...]
    e1 = jnp.exp(l1 - jnp.max(l1, axis=1, keepdims=True))
    out_cls = e1 / jnp.sum(e1, axis=1, keepdims=True)
    cls_ref[...] = out_cls

    ctm = ctm_ref[...]
    tid = jax.lax.broadcasted_iota(jnp.int32, (T, C), 0)
    g = (ctm == tid).astype(jnp.bfloat16)
    ctw = jnp.dot(out_type.astype(jnp.bfloat16), g,
                  preferred_element_type=jnp.float32)
    final_ref[...] = out_cls * (ctw + 1e-8)


@functools.partial(jax.jit, static_argnames=())
def kernel(x, W1, b1, W2, b2, class_type_map):
    b1r = b1.reshape(1, C)
    b2r = b2.reshape(1, T)
    ctm = class_type_map.reshape(1, C)
    grid = (B // BM,)
    out = pl.pallas_call(
        _fused_kernel,
        grid=grid,
        in_specs=[
            pl.BlockSpec((BM, D), lambda i: (i, 0)),
            pl.BlockSpec((D, C), lambda i: (0, 0)),
            pl.BlockSpec((1, C), lambda i: (0, 0)),
            pl.BlockSpec((D, T), lambda i: (0, 0)),
            pl.BlockSpec((1, T), lambda i: (0, 0)),
            pl.BlockSpec((1, C), lambda i: (0, 0)),
        ],
        out_specs=[
            pl.BlockSpec((BM, C), lambda i: (i, 0)),
            pl.BlockSpec((BM, C), lambda i: (i, 0)),
            pl.BlockSpec((BM, T), lambda i: (i, 0)),
        ],
        out_shape=[
            jax.ShapeDtypeStruct((B, C), jnp.float32),
            jax.ShapeDtypeStruct((B, C), jnp.float32),
            jax.ShapeDtypeStruct((B, T), jnp.float32),
        ],
    )(x, W1, b1r, W2, b2r, ctm)
    return (out[0], out[1], out[2])
